# Initial kernel scaffold; baseline (speedup 1.0000x reference)
import jax
import jax.numpy as jnp
from jax.experimental import pallas as pl


def kernel(x_stroke, x_brep, prev_sketch_strokes, W_int, W_tp, W_rep, W_self, b_self, W_local, b_local, W_d1, b_d1, W_d2, b_d2, edge_index_intersects, edge_index_temp_previous, edge_index_represented_by):
    raise NotImplementedError("write your pallas kernel here")


# speedup vs baseline: 6.0484x; 6.0484x over previous
import functools

import jax
import jax.numpy as jnp
from jax import lax
from jax.experimental import pallas as pl
from jax.experimental.pallas import tpu as pltpu
from jax.experimental.pallas import tpu_sc as plsc

NC = 2
NSC = 16
NW = NC * NSC
K = 128
IB = 8
QUANT = NW * K * IB

AUXW = 16


def _pad_edges(edge_index, n_scrap_base):
    e = edge_index.shape[1]
    epad = -(-e // QUANT) * QUANT
    pad = epad - e
    src = jnp.concatenate([edge_index[0], jnp.zeros((pad,), jnp.int32)])
    scrap = n_scrap_base + (jnp.arange(pad, dtype=jnp.int32) % 16)
    dst = jnp.concatenate([edge_index[1], scrap])
    return src.reshape(epad // K, K), dst.reshape(epad // K, K), epad // K


def _sc_segment_sums(ns_rows, c_int, c_tp, c_rep,
                     x_stroke, x_brep, auxt_int, auxt_tp, const_rep,
                     zeros_m, zeros_a, si, di, st, dt, sr, dr):
    rows_per_tile = ns_rows // NSC
    mesh = plsc.VectorSubcoreMesh(core_axis_name="c", subcore_axis_name="s")
    params = pltpu.CompilerParams(use_tc_tiling_on_sc=False)

    def edge_loop(src_h, dst_h, n_chunks, wid, idx_s, idx_d, chunk_fn):
        cpt = n_chunks // NW
        ch_base = wid * cpt

        @pl.loop(0, cpt // IB)
        def _blk(bi):
            ch0 = ch_base + bi * IB
            pltpu.sync_copy(src_h.at[pl.ds(ch0, IB)], idx_s)
            pltpu.sync_copy(dst_h.at[pl.ds(ch0, IB)], idx_d)

            @pl.loop(0, IB)
            def _chunk(j):
                chunk_fn(idx_s.at[j], idx_d.at[j])

    def body_a(x_s, x_b, si_h, di_h, st_h, dt_h, sr_h, dr_h, zm, m_out,
               acc_m, idx_s, idx_d, rows, sem):
        cid = lax.axis_index("c")
        sid = lax.axis_index("s")
        wid = cid * NSC + sid
        r0 = sid * rows_per_tile

        def run_relation(rel, src_h, dst_h, tab_h, n_chunks):
            pltpu.sync_copy(zm, acc_m.at[pl.ds(r0, rows_per_tile)])
            plsc.subcore_barrier()

            def chunk(is_ref, id_ref):
                pltpu.async_copy(tab_h.at[is_ref], rows, sem).wait()
                pltpu.sync_copy(rows, acc_m.at[id_ref], add=True)

            edge_loop(src_h, dst_h, n_chunks, wid, idx_s, idx_d, chunk)
            plsc.subcore_barrier()
            pltpu.sync_copy(acc_m.at[pl.ds(r0, rows_per_tile)],
                            m_out.at[2 * rel + cid, pl.ds(r0, rows_per_tile)])

        run_relation(0, si_h, di_h, x_s, c_int)
        run_relation(1, st_h, dt_h, x_s, c_tp)
        run_relation(2, sr_h, dr_h, x_b, c_rep)

    def body_b(at_i, at_t, c_rep_rows, si_h, di_h, st_h, dt_h, dr_h, za,
               a_out, acc_a, idx_s, idx_d, arows, sem):
        cid = lax.axis_index("c")
        sid = lax.axis_index("s")
        wid = cid * NSC + sid
        r0 = sid * rows_per_tile

        pltpu.sync_copy(za, acc_a.at[pl.ds(r0, rows_per_tile)])
        plsc.subcore_barrier()

        def chunk_gather(tab_h):
            def chunk(is_ref, id_ref):
                pltpu.async_copy(tab_h.at[is_ref], arows, sem).wait()
                pltpu.sync_copy(arows, acc_a.at[id_ref], add=True)
            return chunk

        edge_loop(si_h, di_h, c_int, wid, idx_s, idx_d, chunk_gather(at_i))
        edge_loop(st_h, dt_h, c_tp, wid, idx_s, idx_d, chunk_gather(at_t))
        pltpu.sync_copy(c_rep_rows, arows)

        def chunk_const(is_ref, id_ref):
            pltpu.sync_copy(arows, acc_a.at[id_ref], add=True)

        edge_loop(dr_h, dr_h, c_rep, wid, idx_s, idx_d, chunk_const)
        plsc.subcore_barrier()
        pltpu.sync_copy(acc_a.at[pl.ds(r0, rows_per_tile)],
                        a_out.at[cid, pl.ds(r0, rows_per_tile)])

    fa = pl.kernel(
        body_a,
        out_type=jax.ShapeDtypeStruct((6, ns_rows, 32), jnp.float32),
        mesh=mesh,
        compiler_params=params,
        scratch_types=[
            pltpu.VMEM_SHARED((ns_rows, 32), jnp.float32),
            pltpu.VMEM((IB, K), jnp.int32),
            pltpu.VMEM((IB, K), jnp.int32),
            pltpu.VMEM((K, 32), jnp.float32),
            pltpu.SemaphoreType.DMA,
        ],
    )
    fb = pl.kernel(
        body_b,
        out_type=jax.ShapeDtypeStruct((2, ns_rows, AUXW), jnp.float32),
        mesh=mesh,
        compiler_params=params,
        scratch_types=[
            pltpu.VMEM_SHARED((ns_rows, AUXW), jnp.float32),
            pltpu.VMEM((IB, K), jnp.int32),
            pltpu.VMEM((IB, K), jnp.int32),
            pltpu.VMEM((K, AUXW), jnp.float32),
            pltpu.SemaphoreType.DMA,
        ],
    )
    m_parts = fa(x_stroke, x_brep, si, di, st, dt, sr, dr, zeros_m)
    a_parts = fb(auxt_int, auxt_tp, const_rep, si, di, st, dt, dr, zeros_a)
    return m_parts, a_parts


def _tc_dense(ns_rows, x_stroke, prev, m_parts, a_parts,
              W_int, W_tp, W_rep, W_self, b_self,
              W_local, b_local, W_d1, b_d1, W_d2, b_d2):
    n_blocks = 48
    R = ns_rows // n_blocks

    def body(x_ref, p_ref, m_ref, a_ref,
             wi_ref, wt_ref, wr_ref, ws_ref, bs_ref,
             wl_ref, bl_ref, w1_ref, b1_ref, w2_ref, b2_ref, o_ref):
        x = x_ref[...]
        p = p_ref[...]
        s33 = jnp.concatenate([x, p], axis=1)
        a = a_ref[0] + a_ref[1]

        def rel(i, w, pcol, cnt):
            m = m_ref[2 * i] + m_ref[2 * i + 1]
            v33 = jnp.concatenate([m, pcol], axis=1)
            if cnt is not None:
                v33 = v33 / jnp.maximum(cnt, 1.0)
            return jnp.dot(v33, w, preferred_element_type=jnp.float32)

        agg = (rel(0, wi_ref[...], a[:, 0:1], a[:, 1:2])
               + rel(1, wt_ref[...], a[:, 2:3], None)
               + rel(2, wr_ref[...], jnp.zeros_like(p), a[:, 4:5])
               + jnp.dot(s33, ws_ref[...], preferred_element_type=jnp.float32)
               + bs_ref[...])
        s_out = jax.nn.relu(agg) + s33
        feats = jnp.dot(s_out, wl_ref[...],
                        preferred_element_type=jnp.float32) + bl_ref[...]
        h = jax.nn.relu(jnp.dot(feats, w1_ref[...],
                                preferred_element_type=jnp.float32) + b1_ref[...])
        logits = jnp.dot(h, w2_ref[...],
                         preferred_element_type=jnp.float32) + b2_ref[...]
        o_ref[...] = jax.nn.sigmoid(logits)

    full = lambda s: pl.BlockSpec(s, lambda i: tuple(0 for _ in s))
    grid_spec = pl.GridSpec(
        grid=(n_blocks,),
        in_specs=[
            pl.BlockSpec((R, 32), lambda i: (i, 0)),
            pl.BlockSpec((R, 1), lambda i: (i, 0)),
            pl.BlockSpec((6, R, 32), lambda i: (0, i, 0)),
            pl.BlockSpec((2, R, AUXW), lambda i: (0, i, 0)),
            full((33, 33)), full((33, 33)), full((33, 33)), full((33, 33)),
            full((33,)),
            full((33, 64)), full((64,)),
            full((64, 64)), full((64,)),
            full((64, 1)), full((1,)),
        ],
        out_specs=pl.BlockSpec((R, 1), lambda i: (i, 0)),
    )
    return pl.pallas_call(
        body,
        grid_spec=grid_spec,
        out_shape=jax.ShapeDtypeStruct((ns_rows, 1), jnp.float32),
    )(x_stroke, prev, m_parts, a_parts,
      W_int, W_tp, W_rep, W_self, b_self,
      W_local, b_local, W_d1, b_d1, W_d2, b_d2)


def kernel(x_stroke, x_brep, prev_sketch_strokes, W_int, W_tp, W_rep, W_self,
           b_self, W_local, b_local, W_d1, b_d1, W_d2, b_d2,
           edge_index_intersects, edge_index_temp_previous,
           edge_index_represented_by):
    ns = x_stroke.shape[0]
    ns_rows = -(-(ns + 16) // 384) * 384
    prev = prev_sketch_strokes.astype(jnp.float32)

    ones = jnp.ones((ns, 1), jnp.float32)
    z = jnp.zeros((ns, 1), jnp.float32)
    auxt_int = jnp.concatenate(
        [prev, ones] + [z] * (AUXW - 2), axis=1)
    auxt_tp = jnp.concatenate(
        [z, z, prev, ones] + [z] * (AUXW - 4), axis=1)
    const_rep = jnp.tile(
        jnp.array([[0.0] * 4 + [1.0] + [0.0] * (AUXW - 5)], jnp.float32),
        (K, 1))
    zeros_m = jnp.zeros((ns_rows // NSC, 32), jnp.float32)
    zeros_a = jnp.zeros((ns_rows // NSC, AUXW), jnp.float32)

    si, di, c_int = _pad_edges(edge_index_intersects, ns)
    st, dt, c_tp = _pad_edges(edge_index_temp_previous, ns)
    sr, dr, c_rep = _pad_edges(edge_index_represented_by, ns)

    m_parts, a_parts = _sc_segment_sums(
        ns_rows, c_int, c_tp, c_rep,
        x_stroke, x_brep, auxt_int, auxt_tp, const_rep, zeros_m, zeros_a,
        si, di, st, dt, sr, dr)

    x_pad = jnp.zeros((ns_rows, 32), jnp.float32).at[:ns].set(x_stroke)
    p_pad = jnp.zeros((ns_rows, 1), jnp.float32).at[:ns].set(prev)

    out = _tc_dense(ns_rows, x_pad, p_pad, m_parts, a_parts,
                    W_int, W_tp, W_rep, W_self, b_self,
                    W_local, b_local, W_d1, b_d1, W_d2, b_d2)
    return out[:ns]


# speedup vs baseline: 7.0701x; 1.1689x over previous
import functools

import jax
import jax.numpy as jnp
from jax import lax
from jax.experimental import pallas as pl
from jax.experimental.pallas import tpu as pltpu
from jax.experimental.pallas import tpu_sc as plsc

NC = 2
NSC = 16
NW = NC * NSC
K = 128
IB = 8
QUANT = NW * K * IB

AUXW = 16


def _pad_edges(edge_index, n_scrap_base):
    e = edge_index.shape[1]
    epad = -(-e // QUANT) * QUANT
    pad = epad - e
    src = jnp.concatenate([edge_index[0], jnp.zeros((pad,), jnp.int32)])
    scrap = n_scrap_base + (jnp.arange(pad, dtype=jnp.int32) % 16)
    dst = jnp.concatenate([edge_index[1], scrap])
    return src.reshape(epad // K, K), dst.reshape(epad // K, K), epad // K


def _sc_segment_sums(ns_rows, c_int, c_tp, c_rep,
                     x_stroke, x_brep, auxt_int, auxt_tp, const_rep,
                     zeros_m, zeros_a, si, di, st, dt, sr, dr):
    rows_per_tile = ns_rows // NSC
    mesh = plsc.VectorSubcoreMesh(core_axis_name="c", subcore_axis_name="s")
    params = pltpu.CompilerParams(use_tc_tiling_on_sc=False)

    def edge_loop(src_h, dst_h, n_chunks, wid, idx_s, idx_d, isem,
                  tab_h, rows, gsem, acc, ssem):
        cpt = n_chunks // NW
        ch_base = wid * cpt
        nblk = cpt // IB

        def start_idx(bi, slot):
            ch0 = ch_base + bi * IB
            if tab_h is not None:
                pltpu.async_copy(src_h.at[pl.ds(ch0, IB)], idx_s.at[slot], isem)
            pltpu.async_copy(dst_h.at[pl.ds(ch0, IB)], idx_d.at[slot], isem)

        def wait_idx(slot):
            if tab_h is not None:
                pltpu.make_async_copy(src_h.at[pl.ds(0, IB)],
                                      idx_s.at[slot], isem).wait()
            pltpu.make_async_copy(dst_h.at[pl.ds(0, IB)],
                                  idx_d.at[slot], isem).wait()

        start_idx(0, 0)

        @pl.loop(0, nblk)
        def _blk(bi):
            cur = lax.rem(bi, 2)
            wait_idx(cur)

            @pl.when(bi + 1 < nblk)
            def _():
                start_idx(bi + 1, 1 - cur)

            if tab_h is not None:
                g_desc = pltpu.async_copy(tab_h.at[idx_s.at[cur, 0]],
                                          rows[0], gsem[0])
                s_descs = {}
                for j in range(IB):
                    b = j & 1
                    g_desc.wait()
                    if j + 1 < IB:
                        if j >= 1:
                            s_descs[j - 1].wait()
                        g_desc = pltpu.async_copy(
                            tab_h.at[idx_s.at[cur, j + 1]],
                            rows[1 - b], gsem[1 - b])
                    s_descs[j] = pltpu.async_copy(
                        rows[b], acc.at[idx_d.at[cur, j]], ssem[b], add=True)
                s_descs[IB - 2].wait()
                s_descs[IB - 1].wait()
            else:
                s_descs = [pltpu.async_copy(rows[0],
                                            acc.at[idx_d.at[cur, j]],
                                            ssem[0], add=True)
                           for j in range(IB)]
                for d in s_descs:
                    d.wait()

    def body_a(x_s, x_b, si_h, di_h, st_h, dt_h, sr_h, dr_h, zm, m_out,
               acc_m, idx_s, idx_d, rows0, rows1, isem, gsem0, gsem1,
               ssem0, ssem1):
        cid = lax.axis_index("c")
        sid = lax.axis_index("s")
        wid = cid * NSC + sid
        r0 = sid * rows_per_tile

        def run_relation(rel, src_h, dst_h, tab_h, n_chunks):
            pltpu.sync_copy(zm, acc_m.at[pl.ds(r0, rows_per_tile)])
            plsc.subcore_barrier()
            edge_loop(src_h, dst_h, n_chunks, wid, idx_s, idx_d, isem,
                      tab_h, (rows0, rows1), (gsem0, gsem1),
                      acc_m, (ssem0, ssem1))
            plsc.subcore_barrier()
            pltpu.sync_copy(acc_m.at[pl.ds(r0, rows_per_tile)],
                            m_out.at[2 * rel + cid, pl.ds(r0, rows_per_tile)])

        run_relation(0, si_h, di_h, x_s, c_int)
        run_relation(1, st_h, dt_h, x_s, c_tp)
        run_relation(2, sr_h, dr_h, x_b, c_rep)

    def body_b(at_i, at_t, c_rep_rows, si_h, di_h, st_h, dt_h, dr_h, za,
               a_out, acc_a, idx_s, idx_d, arows0, arows1, isem,
               gsem0, gsem1, ssem0, ssem1):
        cid = lax.axis_index("c")
        sid = lax.axis_index("s")
        wid = cid * NSC + sid
        r0 = sid * rows_per_tile

        pltpu.sync_copy(za, acc_a.at[pl.ds(r0, rows_per_tile)])
        plsc.subcore_barrier()
        edge_loop(si_h, di_h, c_int, wid, idx_s, idx_d, isem,
                  at_i, (arows0, arows1), (gsem0, gsem1),
                  acc_a, (ssem0, ssem1))
        edge_loop(st_h, dt_h, c_tp, wid, idx_s, idx_d, isem,
                  at_t, (arows0, arows1), (gsem0, gsem1),
                  acc_a, (ssem0, ssem1))
        pltpu.sync_copy(c_rep_rows, arows0)
        edge_loop(None, dr_h, c_rep, wid, idx_s, idx_d, isem,
                  None, (arows0, arows1), (gsem0, gsem1),
                  acc_a, (ssem0, ssem1))
        plsc.subcore_barrier()
        pltpu.sync_copy(acc_a.at[pl.ds(r0, rows_per_tile)],
                        a_out.at[cid, pl.ds(r0, rows_per_tile)])

    fa = pl.kernel(
        body_a,
        out_type=jax.ShapeDtypeStruct((6, ns_rows, 32), jnp.float32),
        mesh=mesh,
        compiler_params=params,
        scratch_types=[
            pltpu.VMEM_SHARED((ns_rows, 32), jnp.float32),
            pltpu.VMEM((2, IB, K), jnp.int32),
            pltpu.VMEM((2, IB, K), jnp.int32),
            pltpu.VMEM((K, 32), jnp.float32),
            pltpu.VMEM((K, 32), jnp.float32),
            pltpu.SemaphoreType.DMA,
            pltpu.SemaphoreType.DMA,
            pltpu.SemaphoreType.DMA,
            pltpu.SemaphoreType.DMA,
            pltpu.SemaphoreType.DMA,
        ],
    )
    fb = pl.kernel(
        body_b,
        out_type=jax.ShapeDtypeStruct((2, ns_rows, AUXW), jnp.float32),
        mesh=mesh,
        compiler_params=params,
        scratch_types=[
            pltpu.VMEM_SHARED((ns_rows, AUXW), jnp.float32),
            pltpu.VMEM((2, IB, K), jnp.int32),
            pltpu.VMEM((2, IB, K), jnp.int32),
            pltpu.VMEM((K, AUXW), jnp.float32),
            pltpu.VMEM((K, AUXW), jnp.float32),
            pltpu.SemaphoreType.DMA,
            pltpu.SemaphoreType.DMA,
            pltpu.SemaphoreType.DMA,
            pltpu.SemaphoreType.DMA,
            pltpu.SemaphoreType.DMA,
        ],
    )
    m_parts = fa(x_stroke, x_brep, si, di, st, dt, sr, dr, zeros_m)
    a_parts = fb(auxt_int, auxt_tp, const_rep, si, di, st, dt, dr, zeros_a)
    return m_parts, a_parts


def _tc_dense(ns_rows, x_stroke, prev, m_parts, a_parts,
              W_int, W_tp, W_rep, W_self, b_self,
              W_local, b_local, W_d1, b_d1, W_d2, b_d2):
    n_blocks = 48
    R = ns_rows // n_blocks

    def body(x_ref, p_ref, m_ref, a_ref,
             wi_ref, wt_ref, wr_ref, ws_ref, bs_ref,
             wl_ref, bl_ref, w1_ref, b1_ref, w2_ref, b2_ref, o_ref):
        x = x_ref[...]
        p = p_ref[...]
        s33 = jnp.concatenate([x, p], axis=1)
        a = a_ref[0] + a_ref[1]

        def rel(i, w, pcol, cnt):
            m = m_ref[2 * i] + m_ref[2 * i + 1]
            v33 = jnp.concatenate([m, pcol], axis=1)
            if cnt is not None:
                v33 = v33 / jnp.maximum(cnt, 1.0)
            return jnp.dot(v33, w, preferred_element_type=jnp.float32)

        agg = (rel(0, wi_ref[...], a[:, 0:1], a[:, 1:2])
               + rel(1, wt_ref[...], a[:, 2:3], None)
               + rel(2, wr_ref[...], jnp.zeros_like(p), a[:, 4:5])
               + jnp.dot(s33, ws_ref[...], preferred_element_type=jnp.float32)
               + bs_ref[...])
        s_out = jax.nn.relu(agg) + s33
        feats = jnp.dot(s_out, wl_ref[...],
                        preferred_element_type=jnp.float32) + bl_ref[...]
        h = jax.nn.relu(jnp.dot(feats, w1_ref[...],
                                preferred_element_type=jnp.float32) + b1_ref[...])
        logits = jnp.dot(h, w2_ref[...],
                         preferred_element_type=jnp.float32) + b2_ref[...]
        o_ref[...] = jax.nn.sigmoid(logits)

    full = lambda s: pl.BlockSpec(s, lambda i: tuple(0 for _ in s))
    grid_spec = pl.GridSpec(
        grid=(n_blocks,),
        in_specs=[
            pl.BlockSpec((R, 32), lambda i: (i, 0)),
            pl.BlockSpec((R, 1), lambda i: (i, 0)),
            pl.BlockSpec((6, R, 32), lambda i: (0, i, 0)),
            pl.BlockSpec((2, R, AUXW), lambda i: (0, i, 0)),
            full((33, 33)), full((33, 33)), full((33, 33)), full((33, 33)),
            full((33,)),
            full((33, 64)), full((64,)),
            full((64, 64)), full((64,)),
            full((64, 1)), full((1,)),
        ],
        out_specs=pl.BlockSpec((R, 1), lambda i: (i, 0)),
    )
    return pl.pallas_call(
        body,
        grid_spec=grid_spec,
        out_shape=jax.ShapeDtypeStruct((ns_rows, 1), jnp.float32),
    )(x_stroke, prev, m_parts, a_parts,
      W_int, W_tp, W_rep, W_self, b_self,
      W_local, b_local, W_d1, b_d1, W_d2, b_d2)


def kernel(x_stroke, x_brep, prev_sketch_strokes, W_int, W_tp, W_rep, W_self,
           b_self, W_local, b_local, W_d1, b_d1, W_d2, b_d2,
           edge_index_intersects, edge_index_temp_previous,
           edge_index_represented_by):
    ns = x_stroke.shape[0]
    ns_rows = -(-(ns + 16) // 384) * 384
    prev = prev_sketch_strokes.astype(jnp.float32)

    ones = jnp.ones((ns, 1), jnp.float32)
    z = jnp.zeros((ns, 1), jnp.float32)
    auxt_int = jnp.concatenate(
        [prev, ones] + [z] * (AUXW - 2), axis=1)
    auxt_tp = jnp.concatenate(
        [z, z, prev, ones] + [z] * (AUXW - 4), axis=1)
    const_rep = jnp.tile(
        jnp.array([[0.0] * 4 + [1.0] + [0.0] * (AUXW - 5)], jnp.float32),
        (K, 1))
    zeros_m = jnp.zeros((ns_rows // NSC, 32), jnp.float32)
    zeros_a = jnp.zeros((ns_rows // NSC, AUXW), jnp.float32)

    si, di, c_int = _pad_edges(edge_index_intersects, ns)
    st, dt, c_tp = _pad_edges(edge_index_temp_previous, ns)
    sr, dr, c_rep = _pad_edges(edge_index_represented_by, ns)

    m_parts, a_parts = _sc_segment_sums(
        ns_rows, c_int, c_tp, c_rep,
        x_stroke, x_brep, auxt_int, auxt_tp, const_rep, zeros_m, zeros_a,
        si, di, st, dt, sr, dr)

    x_pad = jnp.zeros((ns_rows, 32), jnp.float32).at[:ns].set(x_stroke)
    p_pad = jnp.zeros((ns_rows, 1), jnp.float32).at[:ns].set(prev)

    out = _tc_dense(ns_rows, x_pad, p_pad, m_parts, a_parts,
                    W_int, W_tp, W_rep, W_self, b_self,
                    W_local, b_local, W_d1, b_d1, W_d2, b_d2)
    return out[:ns]


# speedup vs baseline: 8.8660x; 1.2540x over previous
import functools

import jax
import jax.numpy as jnp
from jax import lax
from jax.experimental import pallas as pl
from jax.experimental.pallas import tpu as pltpu
from jax.experimental.pallas import tpu_sc as plsc

NC = 2
NSC = 16
NW = NC * NSC
K = 128
IB = 8
QUANT = NW * K * IB

AUXW = 16


def _pad_edges(edge_index, n_scrap_base, n_scrap):
    e = edge_index.shape[1]
    epad = -(-e // QUANT) * QUANT
    pad = epad - e
    src = jnp.concatenate([edge_index[0], jnp.zeros((pad,), jnp.int32)])
    scrap = n_scrap_base + (jnp.arange(pad, dtype=jnp.int32) % n_scrap)
    dst = jnp.concatenate([edge_index[1], scrap])
    return src.reshape(epad // K, K), dst.reshape(epad // K, K), epad // K


def _sc_segment_sums(ns_rows, c_int, c_tp, c_rep,
                     x_stroke, x_brep, auxt_int, auxt_tp, const_rep,
                     zeros_m, zeros_a, si, di, st, dt, sr, dr):
    rows_per_tile = ns_rows // NSC
    mesh = plsc.VectorSubcoreMesh(core_axis_name="c", subcore_axis_name="s")
    params = pltpu.CompilerParams(use_tc_tiling_on_sc=False)

    def edge_loop(src_h, dst_h, n_chunks, wid, idx_s, idx_d, isem,
                  tab_h, rows, gsem, acc, ssem):
        nblk = n_chunks // NW // IB

        def start_idx(bi, slot):
            ch0 = (bi * NW + wid) * IB
            if tab_h is not None:
                pltpu.async_copy(src_h.at[pl.ds(ch0, IB)], idx_s.at[slot], isem)
            pltpu.async_copy(dst_h.at[pl.ds(ch0, IB)], idx_d.at[slot], isem)

        def wait_idx(slot):
            if tab_h is not None:
                pltpu.make_async_copy(src_h.at[pl.ds(0, IB)],
                                      idx_s.at[slot], isem).wait()
            pltpu.make_async_copy(dst_h.at[pl.ds(0, IB)],
                                  idx_d.at[slot], isem).wait()

        start_idx(0, 0)
        nbuf = len(rows)

        @pl.loop(0, nblk)
        def _blk(bi):
            cur = lax.rem(bi, 2)
            wait_idx(cur)

            @pl.when(bi + 1 < nblk)
            def _():
                start_idx(bi + 1, 1 - cur)

            if tab_h is not None:
                g_descs = {}
                s_descs = {}
                for j in range(min(nbuf - 1, IB)):
                    g_descs[j] = pltpu.async_copy(
                        tab_h.at[idx_s.at[cur, j]], rows[j % nbuf],
                        gsem[j % nbuf])
                for j in range(IB):
                    b = j % nbuf
                    g_descs[j].wait()
                    nxt = j + nbuf - 1
                    if nxt < IB:
                        if j >= 1:
                            s_descs[j - 1].wait()
                        g_descs[nxt] = pltpu.async_copy(
                            tab_h.at[idx_s.at[cur, nxt]],
                            rows[nxt % nbuf], gsem[nxt % nbuf])
                    s_descs[j] = pltpu.async_copy(
                        rows[b], acc.at[idx_d.at[cur, j]], ssem[b], add=True)
                for j in range(max(0, IB - nbuf), IB):
                    s_descs[j].wait()
            else:
                s_descs = [pltpu.async_copy(rows[0],
                                            acc.at[idx_d.at[cur, j]],
                                            ssem[0], add=True)
                           for j in range(IB)]
                for d in s_descs:
                    d.wait()

    def body_a(x_s, x_b, si_h, di_h, st_h, dt_h, sr_h, dr_h, zm, m_out,
               acc_m, idx_s, idx_d, rows0, rows1, rows2, isem,
               gsem0, gsem1, gsem2, ssem0, ssem1, ssem2):
        cid = lax.axis_index("c")
        sid = lax.axis_index("s")
        wid = cid * NSC + sid
        r0 = sid * rows_per_tile

        def run_relation(rel, src_h, dst_h, tab_h, n_chunks):
            pltpu.sync_copy(zm, acc_m.at[pl.ds(r0, rows_per_tile)])
            plsc.subcore_barrier()
            edge_loop(src_h, dst_h, n_chunks, wid, idx_s, idx_d, isem,
                      tab_h, (rows0, rows1, rows2), (gsem0, gsem1, gsem2),
                      acc_m, (ssem0, ssem1, ssem2))
            plsc.subcore_barrier()
            pltpu.sync_copy(acc_m.at[pl.ds(r0, rows_per_tile)],
                            m_out.at[2 * rel + cid, pl.ds(r0, rows_per_tile)])

        run_relation(0, si_h, di_h, x_s, c_int)
        run_relation(1, st_h, dt_h, x_s, c_tp)
        run_relation(2, sr_h, dr_h, x_b, c_rep)

    def body_b(at_i, at_t, c_rep_rows, si_h, di_h, st_h, dt_h, dr_h, za,
               a_out, acc_a, idx_s, idx_d, arows0, arows1, arows2, isem,
               gsem0, gsem1, gsem2, ssem0, ssem1, ssem2):
        cid = lax.axis_index("c")
        sid = lax.axis_index("s")
        wid = cid * NSC + sid
        r0 = sid * rows_per_tile

        pltpu.sync_copy(za, acc_a.at[pl.ds(r0, rows_per_tile)])
        plsc.subcore_barrier()
        edge_loop(si_h, di_h, c_int, wid, idx_s, idx_d, isem,
                  at_i, (arows0, arows1, arows2), (gsem0, gsem1, gsem2),
                  acc_a, (ssem0, ssem1, ssem2))
        edge_loop(st_h, dt_h, c_tp, wid, idx_s, idx_d, isem,
                  at_t, (arows0, arows1, arows2), (gsem0, gsem1, gsem2),
                  acc_a, (ssem0, ssem1, ssem2))
        pltpu.sync_copy(c_rep_rows, arows0)
        edge_loop(None, dr_h, c_rep, wid, idx_s, idx_d, isem,
                  None, (arows0, arows1, arows2), (gsem0, gsem1, gsem2),
                  acc_a, (ssem0, ssem1, ssem2))
        plsc.subcore_barrier()
        pltpu.sync_copy(acc_a.at[pl.ds(r0, rows_per_tile)],
                        a_out.at[cid, pl.ds(r0, rows_per_tile)])

    fa = pl.kernel(
        body_a,
        out_type=jax.ShapeDtypeStruct((6, ns_rows, 32), jnp.float32),
        mesh=mesh,
        compiler_params=params,
        scratch_types=[
            pltpu.VMEM_SHARED((ns_rows, 32), jnp.float32),
            pltpu.VMEM((2, IB, K), jnp.int32),
            pltpu.VMEM((2, IB, K), jnp.int32),
            pltpu.VMEM((K, 32), jnp.float32),
            pltpu.VMEM((K, 32), jnp.float32),
            pltpu.VMEM((K, 32), jnp.float32),
            pltpu.SemaphoreType.DMA,
            pltpu.SemaphoreType.DMA,
            pltpu.SemaphoreType.DMA,
            pltpu.SemaphoreType.DMA,
            pltpu.SemaphoreType.DMA,
            pltpu.SemaphoreType.DMA,
            pltpu.SemaphoreType.DMA,
        ],
    )
    fb = pl.kernel(
        body_b,
        out_type=jax.ShapeDtypeStruct((2, ns_rows, AUXW), jnp.float32),
        mesh=mesh,
        compiler_params=params,
        scratch_types=[
            pltpu.VMEM_SHARED((ns_rows, AUXW), jnp.float32),
            pltpu.VMEM((2, IB, K), jnp.int32),
            pltpu.VMEM((2, IB, K), jnp.int32),
            pltpu.VMEM((K, AUXW), jnp.float32),
            pltpu.VMEM((K, AUXW), jnp.float32),
            pltpu.VMEM((K, AUXW), jnp.float32),
            pltpu.SemaphoreType.DMA,
            pltpu.SemaphoreType.DMA,
            pltpu.SemaphoreType.DMA,
            pltpu.SemaphoreType.DMA,
            pltpu.SemaphoreType.DMA,
            pltpu.SemaphoreType.DMA,
            pltpu.SemaphoreType.DMA,
        ],
    )
    m_parts = fa(x_stroke, x_brep, si, di, st, dt, sr, dr, zeros_m)
    a_parts = fb(auxt_int, auxt_tp, const_rep, si, di, st, dt, dr, zeros_a)
    return m_parts, a_parts


def _tc_dense(ns_rows, x_stroke, prev, m_parts, a_parts,
              W_int, W_tp, W_rep, W_self, b_self,
              W_local, b_local, W_d1, b_d1, W_d2, b_d2):
    n_blocks = 48
    R = ns_rows // n_blocks

    def body(x_ref, p_ref, m_ref, a_ref,
             wi_ref, wt_ref, wr_ref, ws_ref, bs_ref,
             wl_ref, bl_ref, w1_ref, b1_ref, w2_ref, b2_ref, o_ref):
        x = x_ref[...]
        p = p_ref[...]
        s33 = jnp.concatenate([x, p], axis=1)
        a = a_ref[0] + a_ref[1]

        def rel(i, w, pcol, cnt):
            m = m_ref[2 * i] + m_ref[2 * i + 1]
            v33 = jnp.concatenate([m, pcol], axis=1)
            if cnt is not None:
                v33 = v33 / jnp.maximum(cnt, 1.0)
            return jnp.dot(v33, w, preferred_element_type=jnp.float32)

        agg = (rel(0, wi_ref[...], a[:, 0:1], a[:, 1:2])
               + rel(1, wt_ref[...], a[:, 2:3], None)
               + rel(2, wr_ref[...], jnp.zeros_like(p), a[:, 4:5])
               + jnp.dot(s33, ws_ref[...], preferred_element_type=jnp.float32)
               + bs_ref[...])
        s_out = jax.nn.relu(agg) + s33
        feats = jnp.dot(s_out, wl_ref[...],
                        preferred_element_type=jnp.float32) + bl_ref[...]
        h = jax.nn.relu(jnp.dot(feats, w1_ref[...],
                                preferred_element_type=jnp.float32) + b1_ref[...])
        logits = jnp.dot(h, w2_ref[...],
                         preferred_element_type=jnp.float32) + b2_ref[...]
        o_ref[...] = jax.nn.sigmoid(logits)

    full = lambda s: pl.BlockSpec(s, lambda i: tuple(0 for _ in s))
    grid_spec = pl.GridSpec(
        grid=(n_blocks,),
        in_specs=[
            pl.BlockSpec((R, 32), lambda i: (i, 0)),
            pl.BlockSpec((R, 1), lambda i: (i, 0)),
            pl.BlockSpec((6, R, 32), lambda i: (0, i, 0)),
            pl.BlockSpec((2, R, AUXW), lambda i: (0, i, 0)),
            full((33, 33)), full((33, 33)), full((33, 33)), full((33, 33)),
            full((33,)),
            full((33, 64)), full((64,)),
            full((64, 64)), full((64,)),
            full((64, 1)), full((1,)),
        ],
        out_specs=pl.BlockSpec((R, 1), lambda i: (i, 0)),
    )
    return pl.pallas_call(
        body,
        grid_spec=grid_spec,
        out_shape=jax.ShapeDtypeStruct((ns_rows, 1), jnp.float32),
    )(x_stroke, prev, m_parts, a_parts,
      W_int, W_tp, W_rep, W_self, b_self,
      W_local, b_local, W_d1, b_d1, W_d2, b_d2)


def kernel(x_stroke, x_brep, prev_sketch_strokes, W_int, W_tp, W_rep, W_self,
           b_self, W_local, b_local, W_d1, b_d1, W_d2, b_d2,
           edge_index_intersects, edge_index_temp_previous,
           edge_index_represented_by):
    ns = x_stroke.shape[0]
    ns_rows = -(-(ns + 16) // 384) * 384
    prev = prev_sketch_strokes.astype(jnp.float32)

    ones = jnp.ones((ns, 1), jnp.float32)
    z = jnp.zeros((ns, 1), jnp.float32)
    auxt_int = jnp.concatenate(
        [prev, ones] + [z] * (AUXW - 2), axis=1)
    auxt_tp = jnp.concatenate(
        [z, z, prev, ones] + [z] * (AUXW - 4), axis=1)
    const_rep = jnp.tile(
        jnp.array([[0.0] * 4 + [1.0] + [0.0] * (AUXW - 5)], jnp.float32),
        (K, 1))
    zeros_m = jnp.zeros((ns_rows // NSC, 32), jnp.float32)
    zeros_a = jnp.zeros((ns_rows // NSC, AUXW), jnp.float32)

    n_scrap = ns_rows - ns
    si, di, c_int = _pad_edges(edge_index_intersects, ns, n_scrap)
    st, dt, c_tp = _pad_edges(edge_index_temp_previous, ns, n_scrap)
    sr, dr, c_rep = _pad_edges(edge_index_represented_by, ns, n_scrap)

    m_parts, a_parts = _sc_segment_sums(
        ns_rows, c_int, c_tp, c_rep,
        x_stroke, x_brep, auxt_int, auxt_tp, const_rep, zeros_m, zeros_a,
        si, di, st, dt, sr, dr)

    x_pad = jnp.zeros((ns_rows, 32), jnp.float32).at[:ns].set(x_stroke)
    p_pad = jnp.zeros((ns_rows, 1), jnp.float32).at[:ns].set(prev)

    out = _tc_dense(ns_rows, x_pad, p_pad, m_parts, a_parts,
                    W_int, W_tp, W_rep, W_self, b_self,
                    W_local, b_local, W_d1, b_d1, W_d2, b_d2)
    return out[:ns]


# speedup vs baseline: 10.1528x; 1.1451x over previous
import functools

import jax
import jax.numpy as jnp
from jax import lax
from jax.experimental import pallas as pl
from jax.experimental.pallas import tpu as pltpu
from jax.experimental.pallas import tpu_sc as plsc

NC = 2
NSC = 16
NW = NC * NSC
K = 128
IB = 8
QUANT = NW * K * IB

AUXW = 16


def _pad_edges(edge_index, n_scrap_base, n_scrap):
    e = edge_index.shape[1]
    epad = -(-e // QUANT) * QUANT
    pad = epad - e
    src = jnp.concatenate([edge_index[0], jnp.zeros((pad,), jnp.int32)])
    scrap = n_scrap_base + (jnp.arange(pad, dtype=jnp.int32) % n_scrap)
    dst = jnp.concatenate([edge_index[1], scrap])
    return src.reshape(epad // K, K), dst.reshape(epad // K, K), epad // K


def _sc_segment_sums(ns_rows, c_int, c_tp, c_rep,
                     x_stroke, x_brep, auxt_int, auxt_tp, const_rep,
                     zeros_m, zeros_a, si, di, st, dt, sr, dr):
    rows_per_tile = ns_rows // NSC
    mesh = plsc.VectorSubcoreMesh(core_axis_name="c", subcore_axis_name="s")
    params = pltpu.CompilerParams(use_tc_tiling_on_sc=False)

    def edge_loop(src_h, dst_h, n_chunks, wid, idx_s, idx_d, isem,
                  tab_h, rows, gsem, acc, ssem):
        nblk = n_chunks // NW // IB

        def start_idx(bi, slot):
            ch0 = (bi * NW + wid) * IB
            if tab_h is not None:
                pltpu.async_copy(src_h.at[pl.ds(ch0, IB)], idx_s.at[slot], isem)
            pltpu.async_copy(dst_h.at[pl.ds(ch0, IB)], idx_d.at[slot], isem)

        def wait_idx(slot):
            if tab_h is not None:
                pltpu.make_async_copy(src_h.at[pl.ds(0, IB)],
                                      idx_s.at[slot], isem).wait()
            pltpu.make_async_copy(dst_h.at[pl.ds(0, IB)],
                                  idx_d.at[slot], isem).wait()

        start_idx(0, 0)
        nbuf = len(rows)

        @pl.loop(0, nblk)
        def _blk(bi):
            cur = lax.rem(bi, 2)
            wait_idx(cur)

            @pl.when(bi + 1 < nblk)
            def _():
                start_idx(bi + 1, 1 - cur)

            if tab_h is not None:
                g_descs = {}
                s_descs = {}
                for j in range(min(nbuf - 1, IB)):
                    g_descs[j] = pltpu.async_copy(
                        tab_h.at[idx_s.at[cur, j]], rows[j % nbuf],
                        gsem[j % nbuf])
                for j in range(IB):
                    b = j % nbuf
                    g_descs[j].wait()
                    nxt = j + nbuf - 1
                    if nxt < IB:
                        if j >= 1:
                            s_descs[j - 1].wait()
                        g_descs[nxt] = pltpu.async_copy(
                            tab_h.at[idx_s.at[cur, nxt]],
                            rows[nxt % nbuf], gsem[nxt % nbuf])
                    s_descs[j] = pltpu.async_copy(
                        rows[b], acc.at[idx_d.at[cur, j]], ssem[b], add=True)
                for j in range(max(0, IB - nbuf), IB):
                    s_descs[j].wait()
            else:
                s_descs = [pltpu.async_copy(rows[0],
                                            acc.at[idx_d.at[cur, j]],
                                            ssem[0], add=True)
                           for j in range(IB)]
                for d in s_descs:
                    d.wait()

    def body_a(x_s, x_b, si_h, di_h, st_h, dt_h, sr_h, dr_h, zm, m_out,
               acc_m, idx_s, idx_d, rows0, rows1, rows2, rows3, isem,
               gsem0, gsem1, gsem2, gsem3, ssem0, ssem1, ssem2, ssem3):
        cid = lax.axis_index("c")
        sid = lax.axis_index("s")
        wid = cid * NSC + sid
        r0 = sid * rows_per_tile

        def run_relation(rel, src_h, dst_h, tab_h, n_chunks):
            pltpu.sync_copy(zm, acc_m.at[pl.ds(r0, rows_per_tile)])
            plsc.subcore_barrier()
            edge_loop(src_h, dst_h, n_chunks, wid, idx_s, idx_d, isem,
                      tab_h, (rows0, rows1, rows2, rows3),
                      (gsem0, gsem1, gsem2, gsem3),
                      acc_m, (ssem0, ssem1, ssem2, ssem3))
            plsc.subcore_barrier()
            pltpu.sync_copy(acc_m.at[pl.ds(r0, rows_per_tile)],
                            m_out.at[2 * rel + cid, pl.ds(r0, rows_per_tile)])

        run_relation(0, si_h, di_h, x_s, c_int)
        run_relation(1, st_h, dt_h, x_s, c_tp)
        run_relation(2, sr_h, dr_h, x_b, c_rep)

    def body_b(at_i, at_t, c_rep_rows, si_h, di_h, st_h, dt_h, dr_h, za,
               a_out, acc_a, idx_s, idx_d, arows0, arows1, arows2, arows3,
               isem, gsem0, gsem1, gsem2, gsem3, ssem0, ssem1, ssem2, ssem3):
        cid = lax.axis_index("c")
        sid = lax.axis_index("s")
        wid = cid * NSC + sid
        r0 = sid * rows_per_tile

        pltpu.sync_copy(za, acc_a.at[pl.ds(r0, rows_per_tile)])
        plsc.subcore_barrier()
        edge_loop(si_h, di_h, c_int, wid, idx_s, idx_d, isem,
                  at_i, (arows0, arows1, arows2, arows3),
                  (gsem0, gsem1, gsem2, gsem3),
                  acc_a, (ssem0, ssem1, ssem2, ssem3))
        edge_loop(st_h, dt_h, c_tp, wid, idx_s, idx_d, isem,
                  at_t, (arows0, arows1, arows2, arows3),
                  (gsem0, gsem1, gsem2, gsem3),
                  acc_a, (ssem0, ssem1, ssem2, ssem3))
        pltpu.sync_copy(c_rep_rows, arows0)
        edge_loop(None, dr_h, c_rep, wid, idx_s, idx_d, isem,
                  None, (arows0, arows1, arows2, arows3),
                  (gsem0, gsem1, gsem2, gsem3),
                  acc_a, (ssem0, ssem1, ssem2, ssem3))
        plsc.subcore_barrier()
        pltpu.sync_copy(acc_a.at[pl.ds(r0, rows_per_tile)],
                        a_out.at[cid, pl.ds(r0, rows_per_tile)])

    fa = pl.kernel(
        body_a,
        out_type=jax.ShapeDtypeStruct((6, ns_rows, 32), jnp.float32),
        mesh=mesh,
        compiler_params=params,
        scratch_types=[
            pltpu.VMEM_SHARED((ns_rows, 32), jnp.float32),
            pltpu.VMEM((2, IB, K), jnp.int32),
            pltpu.VMEM((2, IB, K), jnp.int32),
            pltpu.VMEM((K, 32), jnp.float32),
            pltpu.VMEM((K, 32), jnp.float32),
            pltpu.VMEM((K, 32), jnp.float32),
            pltpu.VMEM((K, 32), jnp.float32),
            pltpu.SemaphoreType.DMA,
            pltpu.SemaphoreType.DMA,
            pltpu.SemaphoreType.DMA,
            pltpu.SemaphoreType.DMA,
            pltpu.SemaphoreType.DMA,
            pltpu.SemaphoreType.DMA,
            pltpu.SemaphoreType.DMA,
            pltpu.SemaphoreType.DMA,
            pltpu.SemaphoreType.DMA,
        ],
    )
    fb = pl.kernel(
        body_b,
        out_type=jax.ShapeDtypeStruct((2, ns_rows, AUXW), jnp.float32),
        mesh=mesh,
        compiler_params=params,
        scratch_types=[
            pltpu.VMEM_SHARED((ns_rows, AUXW), jnp.float32),
            pltpu.VMEM((2, IB, K), jnp.int32),
            pltpu.VMEM((2, IB, K), jnp.int32),
            pltpu.VMEM((K, AUXW), jnp.float32),
            pltpu.VMEM((K, AUXW), jnp.float32),
            pltpu.VMEM((K, AUXW), jnp.float32),
            pltpu.VMEM((K, AUXW), jnp.float32),
            pltpu.SemaphoreType.DMA,
            pltpu.SemaphoreType.DMA,
            pltpu.SemaphoreType.DMA,
            pltpu.SemaphoreType.DMA,
            pltpu.SemaphoreType.DMA,
            pltpu.SemaphoreType.DMA,
            pltpu.SemaphoreType.DMA,
            pltpu.SemaphoreType.DMA,
            pltpu.SemaphoreType.DMA,
        ],
    )
    m_parts = fa(x_stroke, x_brep, si, di, st, dt, sr, dr, zeros_m)
    a_parts = fb(auxt_int, auxt_tp, const_rep, si, di, st, dt, dr, zeros_a)
    return m_parts, a_parts


def _tc_dense(ns, ns_rows, x_stroke, prev, m_parts, a_parts,
              W_int, W_tp, W_rep, W_self, b_self,
              W_local, b_local, W_d1, b_d1, W_d2, b_d2):
    n_blocks = 25
    R = ns // n_blocks

    def body(x_ref, p_ref, m_ref, a_ref,
             wi_ref, wt_ref, wr_ref, ws_ref, bs_ref,
             wl_ref, bl_ref, w1_ref, b1_ref, w2_ref, b2_ref, o_ref):
        x = x_ref[...]
        p = p_ref[...]
        s33 = jnp.concatenate([x, p], axis=1)
        a = a_ref[0] + a_ref[1]

        def rel(i, w, pcol, cnt):
            m = m_ref[2 * i] + m_ref[2 * i + 1]
            v33 = jnp.concatenate([m, pcol], axis=1)
            if cnt is not None:
                v33 = v33 / jnp.maximum(cnt, 1.0)
            return jnp.dot(v33, w, preferred_element_type=jnp.float32)

        agg = (rel(0, wi_ref[...], a[:, 0:1], a[:, 1:2])
               + rel(1, wt_ref[...], a[:, 2:3], None)
               + rel(2, wr_ref[...], jnp.zeros_like(p), a[:, 4:5])
               + jnp.dot(s33, ws_ref[...], preferred_element_type=jnp.float32)
               + bs_ref[...])
        s_out = jax.nn.relu(agg) + s33
        feats = jnp.dot(s_out, wl_ref[...],
                        preferred_element_type=jnp.float32) + bl_ref[...]
        h = jax.nn.relu(jnp.dot(feats, w1_ref[...],
                                preferred_element_type=jnp.float32) + b1_ref[...])
        logits = jnp.dot(h, w2_ref[...],
                         preferred_element_type=jnp.float32) + b2_ref[...]
        o_ref[...] = jax.nn.sigmoid(logits)

    full = lambda s: pl.BlockSpec(s, lambda i: tuple(0 for _ in s))
    grid_spec = pl.GridSpec(
        grid=(n_blocks,),
        in_specs=[
            pl.BlockSpec((R, 32), lambda i: (i, 0)),
            pl.BlockSpec((R, 1), lambda i: (i, 0)),
            pl.BlockSpec((6, R, 32), lambda i: (0, i, 0)),
            pl.BlockSpec((2, R, AUXW), lambda i: (0, i, 0)),
            full((33, 33)), full((33, 33)), full((33, 33)), full((33, 33)),
            full((33,)),
            full((33, 64)), full((64,)),
            full((64, 64)), full((64,)),
            full((64, 1)), full((1,)),
        ],
        out_specs=pl.BlockSpec((R, 1), lambda i: (i, 0)),
    )
    return pl.pallas_call(
        body,
        grid_spec=grid_spec,
        out_shape=jax.ShapeDtypeStruct((ns, 1), jnp.float32),
    )(x_stroke, prev, m_parts, a_parts,
      W_int, W_tp, W_rep, W_self, b_self,
      W_local, b_local, W_d1, b_d1, W_d2, b_d2)


def kernel(x_stroke, x_brep, prev_sketch_strokes, W_int, W_tp, W_rep, W_self,
           b_self, W_local, b_local, W_d1, b_d1, W_d2, b_d2,
           edge_index_intersects, edge_index_temp_previous,
           edge_index_represented_by):
    ns = x_stroke.shape[0]
    ns_rows = -(-(ns + 16) // 128) * 128
    prev = prev_sketch_strokes.astype(jnp.float32)

    ones = jnp.ones((ns, 1), jnp.float32)
    z = jnp.zeros((ns, 1), jnp.float32)
    auxt_int = jnp.concatenate(
        [prev, ones] + [z] * (AUXW - 2), axis=1)
    auxt_tp = jnp.concatenate(
        [z, z, prev, ones] + [z] * (AUXW - 4), axis=1)
    const_rep = jnp.tile(
        jnp.array([[0.0] * 4 + [1.0] + [0.0] * (AUXW - 5)], jnp.float32),
        (K, 1))
    zeros_m = jnp.zeros((ns_rows // NSC, 32), jnp.float32)
    zeros_a = jnp.zeros((ns_rows // NSC, AUXW), jnp.float32)

    n_scrap = ns_rows - ns
    si, di, c_int = _pad_edges(edge_index_intersects, ns, n_scrap)
    st, dt, c_tp = _pad_edges(edge_index_temp_previous, ns, n_scrap)
    sr, dr, c_rep = _pad_edges(edge_index_represented_by, ns, n_scrap)

    m_parts, a_parts = _sc_segment_sums(
        ns_rows, c_int, c_tp, c_rep,
        x_stroke, x_brep, auxt_int, auxt_tp, const_rep, zeros_m, zeros_a,
        si, di, st, dt, sr, dr)

    return _tc_dense(ns, ns_rows, x_stroke, prev, m_parts, a_parts,
                     W_int, W_tp, W_rep, W_self, b_self,
                     W_local, b_local, W_d1, b_d1, W_d2, b_d2)
